# Initial kernel scaffold; baseline (speedup 1.0000x reference)
import jax
import jax.numpy as jnp
from jax.experimental import pallas as pl


def kernel(log_qi, G, sqrt_2rho):
    raise NotImplementedError("write your pallas kernel here")


# speedup vs baseline: 1.1522x; 1.1522x over previous
import functools

import jax
import jax.numpy as jnp
import numpy as np
from jax.experimental import pallas as pl
from jax.experimental.pallas import tpu as pltpu

N_TX = 16
N_SYM = 4
N_SAMP = 64
N_RX = 32
BATCH = 4096
_SYMS = np.array([-3.0, -1.0, 1.0, 3.0], dtype=np.float32) / np.float32(np.sqrt(10.0))
_C = np.float32(1.702)
_TINY = np.float32(np.finfo(np.float32).tiny)
_ONE_MINUS_TINY = np.float32(np.float32(1.0) - _TINY)
_STRIDE_K = BATCH * N_TX * N_SYM
_STRIDE_N = N_TX * N_SYM
_STRIDE_T = N_SYM

_BN = 128
_KC = 8


def _np_threefry2x32(k1, k2, x0, x1):
    rot0 = (13, 15, 26, 6)
    rot1 = (17, 29, 16, 24)
    ks = [np.uint32(k1), np.uint32(k2), np.uint32(0)]
    ks[2] = np.uint32(ks[0] ^ ks[1] ^ np.uint32(0x1BD11BDA))
    with np.errstate(over="ignore"):
        x = [np.uint32(np.uint32(x0) + ks[0]), np.uint32(np.uint32(x1) + ks[1])]
        for i, rots in enumerate((rot0, rot1, rot0, rot1, rot0)):
            for r in rots:
                x[0] = np.uint32(x[0] + x[1])
                x[1] = np.uint32((np.uint32(x[1] << np.uint32(r))
                                  | np.uint32(x[1] >> np.uint32(32 - r))) ^ x[0])
            x[0] = np.uint32(x[0] + ks[(i + 1) % 3])
            x[1] = np.uint32(x[1] + ks[(i + 2) % 3] + np.uint32(i + 1))
    return x[0], x[1]


def _folded_keys():
    out = np.zeros((2, N_TX), dtype=np.uint32)
    for xi in range(N_TX):
        b0, b1 = _np_threefry2x32(np.uint32(0), np.uint32(42), np.uint32(0),
                                  np.uint32(xi))
        out[0, xi], out[1, xi] = b0, b1
    return out

_KEYS = _folded_keys()


def _tf_bits(k1, k2, cnt):
    rot0 = (13, 15, 26, 6)
    rot1 = (17, 29, 16, 24)
    ks0 = k1
    ks1 = k2
    ks2 = ks0 ^ ks1 ^ jnp.uint32(0x1BD11BDA)
    x0 = jnp.zeros_like(cnt) + ks0
    x1 = cnt + ks1
    ks = (ks0, ks1, ks2)
    for i, rots in enumerate((rot0, rot1, rot0, rot1, rot0)):
        for r in rots:
            x0 = x0 + x1
            x1 = ((x1 << jnp.uint32(r)) | (x1 >> jnp.uint32(32 - r))) ^ x0
        x0 = x0 + ks[(i + 1) % 3]
        x1 = x1 + ks[(i + 2) % 3] + jnp.uint32(i + 1)
    return x0 ^ x1


def _log_sigmoid(z):
    return jnp.minimum(z, 0.0) - jnp.log1p(jnp.exp(-jnp.abs(z)))


def _mfs_kernel(keys_ref, lq_ref, g_ref, rho_ref, out_ref, sg_ref, sval_ref):
    n0 = pl.program_id(0) * _BN

    rho = rho_ref[0, :]
    sg_ref[...] = g_ref[...] * rho[None, None, :]

    base_iota = (
        jax.lax.broadcasted_iota(jnp.uint32, (N_TX, _KC, _BN), 0) * jnp.uint32(_STRIDE_T)
        + jax.lax.broadcasted_iota(jnp.uint32, (N_TX, _KC, _BN), 1) * jnp.uint32(_STRIDE_K)
        + jax.lax.broadcasted_iota(jnp.uint32, (N_TX, _KC, _BN), 2) * jnp.uint32(_STRIDE_N)
        + jnp.uint32(_STRIDE_N) * n0.astype(jnp.uint32)
    )

    def xi_body(xi, lqs):
        k1 = keys_ref[0, xi]
        k2 = keys_ref[1, xi]

        def samp_body(kc, carry):
            cnt0 = base_iota + kc.astype(jnp.uint32) * jnp.uint32(_KC * _STRIDE_K)
            best = None
            sym = None
            for s in range(N_SYM):
                bits = _tf_bits(k1, k2, cnt0 + jnp.uint32(s))
                fb = jax.lax.bitcast_convert_type(
                    (bits >> jnp.uint32(9)) | jnp.uint32(0x3F800000), jnp.float32)
                f = fb - jnp.float32(1.0)
                u = jnp.maximum(_TINY, f * _ONE_MINUS_TINY + _TINY)
                g = -jnp.log(-jnp.log(u))
                score = g + lqs[s][:, None, :]
                if s == 0:
                    best = score
                    sym = jnp.full_like(score, _SYMS[0])
                else:
                    take = score > best
                    best = jnp.where(take, score, best)
                    sym = jnp.where(take, jnp.float32(_SYMS[s]), sym)
            sval_ref[:, pl.ds(kc * _KC, _KC), :] = sym
            return carry

        jax.lax.fori_loop(0, N_SAMP // _KC, samp_body, 0, unroll=False)

        sval_xi = sval_ref[xi]
        tmask = (jax.lax.broadcasted_iota(jnp.int32, (N_TX, _BN), 0)
                 == xi)

        def r_body(r, accs):
            sgr = sg_ref[r]
            a_xi = jnp.sum(jnp.where(tmask, sgr, 0.0), axis=0)
            base = jnp.zeros((N_SAMP, _BN), jnp.float32)
            for t in range(N_TX):
                base = base + sval_ref[t] * sgr[t][None, :]
            zb = _C * (base - a_xi[None, :] * sval_xi)
            new = []
            for s in range(N_SYM):
                z = zb + (_C * _SYMS[s]) * a_xi[None, :]
                new.append(accs[s] + _log_sigmoid(z))
            return tuple(new)

        zero = jnp.zeros((N_SAMP, _BN), jnp.float32)
        accs = jax.lax.fori_loop(0, N_RX, r_body, (zero, zero, zero, zero),
                                 unroll=False)

        new_lqs = []
        for s in range(N_SYM):
            ex = jnp.sum(accs[s], axis=0) * jnp.float32(1.0 / N_SAMP)
            new_lqs.append(jnp.where(tmask, ex[None, :], lqs[s]))
        m = jnp.maximum(jnp.maximum(new_lqs[0], new_lqs[1]),
                        jnp.maximum(new_lqs[2], new_lqs[3]))
        return tuple(v - m for v in new_lqs)

    lqs0 = tuple(lq_ref[s] for s in range(N_SYM))
    lqs = jax.lax.fori_loop(0, N_TX, xi_body, lqs0, unroll=False)
    for s in range(N_SYM):
        out_ref[s] = lqs[s]


@functools.partial(jax.jit, static_argnums=())
def _run(lqt, gt, rho2d, keys):
    grid = (BATCH // _BN,)
    return pl.pallas_call(
        _mfs_kernel,
        grid=grid,
        in_specs=[
            pl.BlockSpec(memory_space=pltpu.SMEM),
            pl.BlockSpec((N_SYM, N_TX, _BN), lambda i: (0, 0, i)),
            pl.BlockSpec((N_RX, N_TX, _BN), lambda i: (0, 0, i)),
            pl.BlockSpec((1, _BN), lambda i: (0, i)),
        ],
        out_specs=pl.BlockSpec((N_SYM, N_TX, _BN), lambda i: (0, 0, i)),
        out_shape=jax.ShapeDtypeStruct((N_SYM, N_TX, BATCH), jnp.float32),
        scratch_shapes=[
            pltpu.VMEM((N_RX, N_TX, _BN), jnp.float32),
            pltpu.VMEM((N_TX, N_SAMP, _BN), jnp.float32),
        ],
        compiler_params=pltpu.CompilerParams(
            dimension_semantics=("arbitrary",),
        ),
    )(keys, lqt, gt, rho2d)


def kernel(log_qi, G, sqrt_2rho):
    lqt = jnp.transpose(log_qi, (2, 1, 0))
    gt = jnp.transpose(G, (1, 2, 0))
    rho2d = jnp.reshape(sqrt_2rho, (1, BATCH))
    keys = jnp.asarray(_KEYS)
    out = _run(lqt, gt, rho2d, keys)
    return jnp.transpose(out, (2, 1, 0))
